# Initial kernel scaffold; baseline (speedup 1.0000x reference)
import jax
import jax.numpy as jnp
from jax.experimental import pallas as pl


def kernel(x, emb, W1, b1, W2, b2):
    raise NotImplementedError("write your pallas kernel here")


# speedup vs baseline: 3.3538x; 3.3538x over previous
import functools

import jax
import jax.numpy as jnp
from jax import lax
from jax.experimental import pallas as pl
from jax.experimental.pallas import tpu as pltpu
from jax.experimental.pallas import tpu_sc as plsc

_NC = 2
_NS = 16
_NW = _NC * _NS
_LANES = 16


def _pool_call(x, emb):
    B, L = x.shape
    _, E = emb.shape
    SPW = B // _NW
    G = 128
    NBUF = 4
    NGRP = SPW // G
    EV = E // _LANES
    L_HI = (L // 128) * 128
    L_LO = L - L_HI
    scale = 1.0 / float(L)

    mesh = plsc.VectorSubcoreMesh(
        core_axis_name="c", subcore_axis_name="s",
        num_cores=_NC, num_subcores=_NS)

    @functools.partial(
        pl.kernel,
        out_type=jax.ShapeDtypeStruct((B, E), jnp.float32),
        mesh=mesh,
        compiler_params=pltpu.CompilerParams(use_tc_tiling_on_sc=False),
        scratch_types=[
            pltpu.VMEM((G, L), jnp.int32),
            pltpu.VMEM((NBUF, L, E), jnp.float32),
            pltpu.VMEM((G, E), jnp.float32),
            pltpu.SemaphoreType.DMA((NBUF,)),
        ],
    )
    def pool(x_hbm, emb_hbm, out_hbm, idx_v, rows_v, out_v, sems):
        wid = lax.axis_index("s") * _NC + lax.axis_index("c")
        base = wid * SPW

        def fire(r, s):
            pltpu.async_copy(
                emb_hbm.at[idx_v.at[s, pl.ds(0, L_HI)]],
                rows_v.at[r, pl.ds(0, L_HI)], sems.at[r])
            pltpu.async_copy(
                emb_hbm.at[idx_v.at[s, pl.ds(L_HI, L_LO)]],
                rows_v.at[r, pl.ds(L_HI, L_LO)], sems.at[r])

        def group_body(g, _):
            g_base = base + g * G
            pltpu.sync_copy(x_hbm.at[pl.ds(g_base, G), :], idx_v)
            for r in range(NBUF):
                fire(r, r)

            def chunk_body(i, _):
                b = i * NBUF
                for r in range(NBUF):
                    s = b + r
                    pltpu.make_async_copy(
                        emb_hbm.at[pl.ds(0, L)], rows_v.at[r], sems.at[r]
                    ).wait()

                    def red_body(it, accs):
                        row = it * 8
                        accs = list(accs)
                        for rr in range(8):
                            for cidx in range(EV):
                                accs[cidx] = accs[cidx] + rows_v[
                                    r, row + rr, pl.ds(cidx * _LANES, _LANES)]
                        return tuple(accs)

                    zero = jnp.zeros((_LANES,), jnp.float32)
                    accs = lax.fori_loop(
                        0, L // 8, red_body, (zero,) * EV, unroll=False)
                    for cidx in range(EV):
                        out_v[s, pl.ds(cidx * _LANES, _LANES)] = (
                            accs[cidx] * scale)

                    ns = s + NBUF

                    @pl.when(ns < G)
                    def _():
                        fire(r, ns)
                return ()

            lax.fori_loop(0, G // NBUF, chunk_body, ())
            pltpu.sync_copy(out_v, out_hbm.at[pl.ds(g_base, G), :])
            return ()

        lax.fori_loop(0, NGRP, group_body, ())

    return pool(x, emb)


def _mlp_kernel(p_ref, w1_ref, b1_ref, w2_ref, b2_ref, o_ref):
    p = p_ref[...]
    h = lax.dot_general(
        p, w1_ref[...], (((1,), (1,)), ((), ())),
        preferred_element_type=jnp.float32,
        precision=lax.Precision.HIGHEST)
    h = jnp.maximum(h + b1_ref[...], 0.0)
    logits = lax.dot_general(
        h, w2_ref[...], (((1,), (1,)), ((), ())),
        preferred_element_type=jnp.float32,
        precision=lax.Precision.HIGHEST) + b2_ref[...]
    m = jnp.max(logits, axis=1, keepdims=True)
    lse = m + jnp.log(jnp.sum(jnp.exp(logits - m), axis=1, keepdims=True))
    o_ref[...] = logits - lse


def _mlp_call(pooled, W1, b1, W2, b2):
    B, E = pooled.shape
    H = W1.shape[0]
    BS = 2048
    return pl.pallas_call(
        _mlp_kernel,
        grid=(B // BS,),
        in_specs=[
            pl.BlockSpec((BS, E), lambda i: (i, 0)),
            pl.BlockSpec((H, E), lambda i: (0, 0)),
            pl.BlockSpec((1, H), lambda i: (0, 0)),
            pl.BlockSpec((2, H), lambda i: (0, 0)),
            pl.BlockSpec((1, 2), lambda i: (0, 0)),
        ],
        out_specs=pl.BlockSpec((BS, 2), lambda i: (i, 0)),
        out_shape=jax.ShapeDtypeStruct((B, 2), jnp.float32),
    )(pooled, W1, b1, W2, b2)


def kernel(x, emb, W1, b1, W2, b2):
    x = x.astype(jnp.int32)
    pooled = _pool_call(x, emb)
    return _mlp_call(pooled, W1, b1.reshape(1, -1), W2, b2.reshape(1, -1))


# speedup vs baseline: 3.5797x; 1.0674x over previous
import functools

import jax
import jax.numpy as jnp
from jax import lax
from jax.experimental import pallas as pl
from jax.experimental.pallas import tpu as pltpu
from jax.experimental.pallas import tpu_sc as plsc

_NC = 2
_NS = 16
_NW = _NC * _NS
_LANES = 16


def _pool_call(x, emb2, B, L, E):
    SPW = B // _NW
    G = 128
    NBUF = 4
    NGRP = SPW // G
    EV = E // _LANES
    L_HI = (L // 128) * 128
    L_LO = L - L_HI
    scale = 1.0 / float(L)

    mesh = plsc.VectorSubcoreMesh(
        core_axis_name="c", subcore_axis_name="s",
        num_cores=_NC, num_subcores=_NS)

    @functools.partial(
        pl.kernel,
        out_type=jax.ShapeDtypeStruct((B, E), jnp.float32),
        mesh=mesh,
        compiler_params=pltpu.CompilerParams(use_tc_tiling_on_sc=False),
        scratch_types=[
            pltpu.VMEM((G, L), jnp.int32),
            pltpu.VMEM((NBUF, L, E), jnp.float32),
            pltpu.VMEM((G, E), jnp.float32),
            pltpu.SemaphoreType.DMA((NBUF,)),
        ],
    )
    def pool(x_hbm, emb_hbm, out_hbm, idx_v, rows_v, out_v, sems):
        wid = lax.axis_index("s") * _NC + lax.axis_index("c")
        base = wid * SPW

        def fire(r, s):
            pltpu.async_copy(
                emb_hbm.at[idx_v.at[s, pl.ds(0, L_HI)]],
                rows_v.at[r, pl.ds(0, L_HI)], sems.at[r])
            pltpu.async_copy(
                emb_hbm.at[idx_v.at[s, pl.ds(L_HI, L_LO)]],
                rows_v.at[r, pl.ds(L_HI, L_LO)], sems.at[r])

        def group_body(g, _):
            g_base = base + g * G
            pltpu.sync_copy(x_hbm.at[pl.ds(g_base, G), :], idx_v)
            for r in range(NBUF):
                fire(r, r)

            def chunk_body(i, _):
                b = i * NBUF
                for r in range(NBUF):
                    s = b + r
                    pltpu.make_async_copy(
                        emb_hbm.at[pl.ds(0, L)], rows_v.at[r], sems.at[r]
                    ).wait()

                    def red_body(it, accs):
                        row = it * 8
                        accs = list(accs)
                        for rr in range(8):
                            for cidx in range(EV):
                                accs[cidx] = accs[cidx] + rows_v[
                                    r, row + rr, pl.ds(cidx * _LANES, _LANES)]
                        return tuple(accs)

                    zero = jnp.zeros((_LANES,), jnp.float32)
                    accs = lax.fori_loop(
                        0, L // 8, red_body, (zero,) * EV, unroll=False)
                    for cidx in range(EV):
                        out_v[s, pl.ds(cidx * _LANES, _LANES)] = (
                            accs[cidx] * scale)

                    ns = s + NBUF

                    @pl.when(ns < G)
                    def _():
                        fire(r, ns)
                return ()

            lax.fori_loop(0, G // NBUF, chunk_body, ())
            pltpu.sync_copy(out_v, out_hbm.at[pl.ds(g_base, G), :])
            return ()

        lax.fori_loop(0, NGRP, group_body, ())

    return pool(x, emb2)


def _mlp_kernel(p_ref, w1_ref, b1_ref, w2_ref, b2_ref, o_ref):
    p = p_ref[...]
    h = lax.dot_general(
        p, w1_ref[...], (((1,), (1,)), ((), ())),
        preferred_element_type=jnp.float32,
        precision=lax.Precision.HIGHEST)
    h = jnp.maximum(h + b1_ref[...], 0.0)
    logits = lax.dot_general(
        h, w2_ref[...], (((1,), (1,)), ((), ())),
        preferred_element_type=jnp.float32,
        precision=lax.Precision.HIGHEST) + b2_ref[...]
    m = jnp.max(logits, axis=1, keepdims=True)
    lse = m + jnp.log(jnp.sum(jnp.exp(logits - m), axis=1, keepdims=True))
    o_ref[...] = logits - lse


def _mlp_call(pooled, W1, b1, W2, b2):
    B, E = pooled.shape
    H = W1.shape[0]
    BS = 2048
    return pl.pallas_call(
        _mlp_kernel,
        grid=(B // BS,),
        in_specs=[
            pl.BlockSpec((BS, E), lambda i: (i, 0)),
            pl.BlockSpec((H, E), lambda i: (0, 0)),
            pl.BlockSpec((1, H), lambda i: (0, 0)),
            pl.BlockSpec((2, H), lambda i: (0, 0)),
            pl.BlockSpec((1, 2), lambda i: (0, 0)),
        ],
        out_specs=pl.BlockSpec((BS, 2), lambda i: (i, 0)),
        out_shape=jax.ShapeDtypeStruct((B, 2), jnp.float32),
    )(pooled, W1, b1, W2, b2)


def kernel(x, emb, W1, b1, W2, b2):
    B, L = x.shape
    V, E = emb.shape
    x2 = x.astype(jnp.int32) * 2
    emb2 = jnp.concatenate([emb, jnp.zeros_like(emb)], axis=1).reshape(2 * V, E)
    pooled = _pool_call(x2, emb2, B, L, E)
    return _mlp_call(pooled, W1, b1.reshape(1, -1), W2, b2.reshape(1, -1))


# speedup vs baseline: 5.3337x; 1.4900x over previous
import functools

import jax
import jax.numpy as jnp
from jax import lax
from jax.experimental import pallas as pl
from jax.experimental.pallas import tpu as pltpu
from jax.experimental.pallas import tpu_sc as plsc

_NC = 2
_NS = 16
_NW = _NC * _NS
_LANES = 16


def _pool_call(x, emb2, B, L, E, XL):
    SPW = B // _NW
    G = 128
    NBUF = 4
    NGRP = SPW // G
    EV = E // _LANES
    L_HI = (L // 128) * 128
    L_LO = L - L_HI
    scale = 1.0 / float(L)

    mesh = plsc.VectorSubcoreMesh(
        core_axis_name="c", subcore_axis_name="s",
        num_cores=_NC, num_subcores=_NS)

    @functools.partial(
        pl.kernel,
        out_type=jax.ShapeDtypeStruct((B, E), jnp.float32),
        mesh=mesh,
        compiler_params=pltpu.CompilerParams(use_tc_tiling_on_sc=False),
        scratch_types=[
            pltpu.VMEM((G, XL), jnp.int32),
            pltpu.VMEM((NBUF, L, E), jnp.float32),
            pltpu.VMEM((G, E), jnp.float32),
            pltpu.SemaphoreType.DMA((NBUF,)),
        ],
    )
    def pool(x_hbm, emb_hbm, out_hbm, idx_v, rows_v, out_v, sems):
        wid = lax.axis_index("s") * _NC + lax.axis_index("c")
        base = wid * SPW

        def fire(r, s):
            pltpu.async_copy(
                emb_hbm.at[idx_v.at[s, pl.ds(0, L_HI)]],
                rows_v.at[r, pl.ds(0, L_HI)], sems.at[r])
            pltpu.async_copy(
                emb_hbm.at[idx_v.at[s, pl.ds(L_HI, L_LO)]],
                rows_v.at[r, pl.ds(L_HI, L_LO)], sems.at[r])

        def group_body(g, _):
            g_base = base + g * G
            pltpu.sync_copy(x_hbm.at[pl.ds(g_base, G), :], idx_v)
            for r in range(NBUF):
                fire(r, r)

            def chunk_body(i, _):
                b = i * NBUF
                for r in range(NBUF):
                    s = b + r
                    pltpu.make_async_copy(
                        emb_hbm.at[pl.ds(0, L)], rows_v.at[r], sems.at[r]
                    ).wait()

                    def red_body(it, accs):
                        row = it * 8
                        accs = list(accs)
                        for rr in range(8):
                            for cidx in range(EV):
                                accs[cidx] = accs[cidx] + rows_v[
                                    r, row + rr, pl.ds(cidx * _LANES, _LANES)]
                        return tuple(accs)

                    zero = jnp.zeros((_LANES,), jnp.float32)
                    accs = lax.fori_loop(
                        0, L // 8, red_body, (zero,) * EV, unroll=False)
                    for cidx in range(EV):
                        out_v[s, pl.ds(cidx * _LANES, _LANES)] = (
                            accs[cidx] * scale)

                    ns = s + NBUF

                    @pl.when(ns < G)
                    def _():
                        fire(r, ns)
                return ()

            lax.fori_loop(0, G // NBUF, chunk_body, ())
            pltpu.sync_copy(out_v, out_hbm.at[pl.ds(g_base, G), :])
            return ()

        lax.fori_loop(0, NGRP, group_body, ())

    return pool(x, emb2)


def _tpad_kernel(e_ref, o_ref):
    E = e_ref.shape[0]
    o_ref[:, :E] = e_ref[...].T
    o_ref[:, E:] = jnp.zeros_like(o_ref[:, E:])


def _interleave_call(emb):
    V, E = emb.shape
    VB = 16384
    out = pl.pallas_call(
        _tpad_kernel,
        grid=(pl.cdiv(V, VB),),
        in_specs=[pl.BlockSpec((E, VB), lambda i: (0, i))],
        out_specs=pl.BlockSpec((VB, 2 * E), lambda i: (i, 0)),
        out_shape=jax.ShapeDtypeStruct((V, 2 * E), jnp.float32),
    )(emb.T)
    return out.reshape(2 * V, E)


def _mlp_kernel(p_ref, w1_ref, b1_ref, w2_ref, b2_ref, o_ref):
    p = p_ref[...]
    h = lax.dot_general(
        p, w1_ref[...], (((1,), (1,)), ((), ())),
        preferred_element_type=jnp.float32,
        precision=lax.Precision.DEFAULT)
    h = jnp.maximum(h + b1_ref[...], 0.0)
    logits = lax.dot_general(
        h, w2_ref[...], (((1,), (1,)), ((), ())),
        preferred_element_type=jnp.float32,
        precision=lax.Precision.DEFAULT) + b2_ref[...]
    m = jnp.max(logits, axis=1, keepdims=True)
    lse = m + jnp.log(jnp.sum(jnp.exp(logits - m), axis=1, keepdims=True))
    o_ref[...] = logits - lse


def _mlp_call(pooled, W1, b1, W2, b2):
    B, E = pooled.shape
    H = W1.shape[0]
    BS = 4096
    return pl.pallas_call(
        _mlp_kernel,
        grid=(B // BS,),
        in_specs=[
            pl.BlockSpec((BS, E), lambda i: (i, 0)),
            pl.BlockSpec((H, E), lambda i: (0, 0)),
            pl.BlockSpec((1, H), lambda i: (0, 0)),
            pl.BlockSpec((2, H), lambda i: (0, 0)),
            pl.BlockSpec((1, 2), lambda i: (0, 0)),
        ],
        out_specs=pl.BlockSpec((BS, 2), lambda i: (i, 0)),
        out_shape=jax.ShapeDtypeStruct((B, 2), jnp.float32),
    )(pooled, W1, b1, W2, b2)


def kernel(x, emb, W1, b1, W2, b2):
    B, L = x.shape
    V, E = emb.shape
    XL = 256
    x2 = jnp.pad(x.astype(jnp.int32) * 2, ((0, 0), (0, XL - L)))
    emb2 = _interleave_call(emb)
    pooled = _pool_call(x2, emb2, B, L, E, XL)
    return _mlp_call(pooled, W1, b1.reshape(1, -1), W2, b2.reshape(1, -1))


# speedup vs baseline: 5.4050x; 1.0134x over previous
import functools

import jax
import jax.numpy as jnp
from jax import lax
from jax.experimental import pallas as pl
from jax.experimental.pallas import tpu as pltpu
from jax.experimental.pallas import tpu_sc as plsc

_NC = 2
_NS = 16
_NW = _NC * _NS
_LANES = 16


def _pool_call(x, emb2, B, L, E, XL):
    SPW = B // _NW
    G = 128
    NBUF = 4
    NGRP = SPW // G
    EV = E // _LANES
    L_HI = (L // 128) * 128
    L_LO = L - L_HI
    scale = 1.0 / float(L)

    mesh = plsc.VectorSubcoreMesh(
        core_axis_name="c", subcore_axis_name="s",
        num_cores=_NC, num_subcores=_NS)

    @functools.partial(
        pl.kernel,
        out_type=jax.ShapeDtypeStruct((B, E), jnp.float32),
        mesh=mesh,
        compiler_params=pltpu.CompilerParams(use_tc_tiling_on_sc=False),
        scratch_types=[
            pltpu.VMEM((G, XL), jnp.int32),
            pltpu.VMEM((NBUF, L, E), jnp.float32),
            pltpu.VMEM((G, E), jnp.float32),
            pltpu.SemaphoreType.DMA((NBUF,)),
        ],
    )
    def pool(x_hbm, emb_hbm, out_hbm, idx_v, rows_v, out_v, sems):
        wid = lax.axis_index("s") * _NC + lax.axis_index("c")
        base = wid * SPW

        def fire(r, s):
            pltpu.async_copy(
                emb_hbm.at[idx_v.at[s, pl.ds(0, L_HI)]],
                rows_v.at[r, pl.ds(0, L_HI)], sems.at[r])
            pltpu.async_copy(
                emb_hbm.at[idx_v.at[s, pl.ds(L_HI, L_LO)]],
                rows_v.at[r, pl.ds(L_HI, L_LO)], sems.at[r])

        def group_body(g, _):
            g_base = base + g * G
            pltpu.sync_copy(x_hbm.at[pl.ds(g_base, G), :], idx_v)
            for r in range(NBUF):
                fire(r, r)

            def chunk_body(i, _):
                b = i * NBUF
                for r in range(NBUF):
                    s = b + r
                    pltpu.make_async_copy(
                        emb_hbm.at[pl.ds(0, L)], rows_v.at[r], sems.at[r]
                    ).wait()

                    def red_body(it, accs):
                        row = it * 8
                        accs = list(accs)
                        for rr in range(8):
                            for cidx in range(EV):
                                accs[cidx] = accs[cidx] + rows_v[
                                    r, row + rr, pl.ds(cidx * _LANES, _LANES)]
                        return tuple(accs)

                    zero = jnp.zeros((_LANES,), jnp.float32)
                    accs = lax.fori_loop(
                        0, L // 8, red_body, (zero,) * EV, unroll=False)
                    for cidx in range(EV):
                        out_v[s, pl.ds(cidx * _LANES, _LANES)] = (
                            accs[cidx] * scale)

                    ns = s + NBUF

                    @pl.when(ns < G)
                    def _():
                        fire(r, ns)
                return ()

            lax.fori_loop(0, G // NBUF, chunk_body, ())
            pltpu.sync_copy(out_v, out_hbm.at[pl.ds(g_base, G), :])
            return ()

        lax.fori_loop(0, NGRP, group_body, ())

    return pool(x, emb2)


def _tpad_kernel(e_ref, o_ref):
    E = e_ref.shape[0]
    o_ref[:, :E] = e_ref[...].T
    o_ref[:, E:] = jnp.zeros_like(o_ref[:, E:])


def _interleave_call(emb):
    V, E = emb.shape
    VB = 32768
    out = pl.pallas_call(
        _tpad_kernel,
        grid=(pl.cdiv(V, VB),),
        in_specs=[pl.BlockSpec((E, VB), lambda i: (0, i))],
        out_specs=pl.BlockSpec((VB, 2 * E), lambda i: (i, 0)),
        out_shape=jax.ShapeDtypeStruct((V, 2 * E), jnp.float32),
    )(emb.T)
    return out.reshape(2 * V, E)


def _mlp_kernel(p_ref, w1_ref, b1_ref, w2_ref, b2_ref, o_ref):
    p = p_ref[...]
    h = lax.dot_general(
        p, w1_ref[...], (((1,), (1,)), ((), ())),
        preferred_element_type=jnp.float32,
        precision=lax.Precision.DEFAULT)
    h = jnp.maximum(h + b1_ref[...], 0.0)
    logits = lax.dot_general(
        h, w2_ref[...], (((1,), (1,)), ((), ())),
        preferred_element_type=jnp.float32,
        precision=lax.Precision.DEFAULT) + b2_ref[...]
    m = jnp.max(logits, axis=1, keepdims=True)
    lse = m + jnp.log(jnp.sum(jnp.exp(logits - m), axis=1, keepdims=True))
    o_ref[...] = logits - lse


def _mlp_call(pooled, W1, b1, W2, b2):
    B, E = pooled.shape
    H = W1.shape[0]
    BS = 4096
    return pl.pallas_call(
        _mlp_kernel,
        grid=(B // BS,),
        in_specs=[
            pl.BlockSpec((BS, E), lambda i: (i, 0)),
            pl.BlockSpec((H, E), lambda i: (0, 0)),
            pl.BlockSpec((1, H), lambda i: (0, 0)),
            pl.BlockSpec((2, H), lambda i: (0, 0)),
            pl.BlockSpec((1, 2), lambda i: (0, 0)),
        ],
        out_specs=pl.BlockSpec((BS, 2), lambda i: (i, 0)),
        out_shape=jax.ShapeDtypeStruct((B, 2), jnp.float32),
    )(pooled, W1, b1, W2, b2)


def kernel(x, emb, W1, b1, W2, b2):
    B, L = x.shape
    V, E = emb.shape
    XL = 256
    x2 = jnp.pad(x.astype(jnp.int32) * 2, ((0, 0), (0, XL - L)))
    emb2 = _interleave_call(emb)
    pooled = _pool_call(x2, emb2, B, L, E, XL)
    return _mlp_call(pooled, W1, b1.reshape(1, -1), W2, b2.reshape(1, -1))
